# baseline (speedup 1.0000x reference)
import functools

import jax
import jax.numpy as jnp
from jax import lax
from jax.experimental import pallas as pl
from jax.experimental.pallas import tpu as pltpu
from jax.experimental.pallas import tpu_sc as plsc

N_NODES = 10000
N_EDGES = 320000
D_IN = 128
D_OUT = 128
DP = 144

NC = 2
NS = 16
NW = NC * NS
EDGE_B = 128
CHUNK = 4
E_PER_W = 10240
E_TOTAL_PAD = E_PER_W * NW
NB = E_PER_W // EDGE_B
NCH = NB // CHUNK
N_PAD = 10240
ROWS_PER_TILE = N_PAD // NS
DUMMY_ROW = N_PAD - 8


@functools.cache
def _build_sc_scatter():
    mesh = plsc.VectorSubcoreMesh(core_axis_name="c", subcore_axis_name="s",
                                  num_cores=NC, num_subcores=NS)

    @functools.partial(
        pl.kernel,
        out_type=jax.ShapeDtypeStruct((NC, N_PAD, DP), jnp.float32),
        mesh=mesh,
        scratch_types=[
            pltpu.VMEM((CHUNK, EDGE_B), jnp.int32),
            pltpu.VMEM((CHUNK, EDGE_B), jnp.int32),
            pltpu.VMEM((EDGE_B, DP), jnp.float32),
            pltpu.VMEM((EDGE_B, DP), jnp.float32),
            pltpu.VMEM_SHARED((N_PAD, DP), jnp.float32),
            pltpu.SemaphoreType.DMA,
            pltpu.SemaphoreType.DMA,
        ],
        compiler_params=pltpu.CompilerParams(use_tc_tiling_on_sc=False),
    )
    def _sc_scatter(xa_hbm, col_hbm, row_hbm, zeros_hbm, out_hbm,
                    colb, rowb, buf_a, buf_b, agg_sh, sem_a, sem_b):
        cid = lax.axis_index("c")
        sid = lax.axis_index("s")
        w = cid * NS + sid
        r0 = sid * ROWS_PER_TILE
        pltpu.sync_copy(zeros_hbm, agg_sh.at[pl.ds(r0, ROWS_PER_TILE)])
        plsc.subcore_barrier()

        pltpu.sync_copy(col_hbm.at[w, pl.ds(0, CHUNK)], colb)
        pltpu.sync_copy(row_hbm.at[w, pl.ds(0, CHUNK)], rowb)
        pltpu.async_copy(xa_hbm.at[colb.at[0]], buf_a, sem_a)

        bufs = (buf_a, buf_b)
        sems = (sem_a, sem_b)

        def chunk_body(c, carry):
            for j in range(CHUNK):
                src, ssem = bufs[j % 2], sems[j % 2]
                nxt, nsem = bufs[(j + 1) % 2], sems[(j + 1) % 2]
                pltpu.make_async_copy(
                    xa_hbm.at[colb.at[j]], src, ssem).wait()
                if j + 1 < CHUNK:
                    pltpu.async_copy(xa_hbm.at[colb.at[j + 1]], nxt, nsem)
                    pltpu.sync_copy(src, agg_sh.at[rowb.at[j]], add=True)
                else:
                    pltpu.sync_copy(src, agg_sh.at[rowb.at[j]], add=True)

                    @pl.when(c < NCH - 1)
                    def _():
                        nxt_chunk = (c + 1) * CHUNK
                        pltpu.sync_copy(
                            col_hbm.at[w, pl.ds(nxt_chunk, CHUNK)], colb)
                        pltpu.sync_copy(
                            row_hbm.at[w, pl.ds(nxt_chunk, CHUNK)], rowb)
                        pltpu.async_copy(xa_hbm.at[colb.at[0]], nxt, nsem)
            return carry

        lax.fori_loop(0, NCH, chunk_body, 0)
        plsc.subcore_barrier()
        pltpu.sync_copy(agg_sh.at[pl.ds(r0, ROWS_PER_TILE)],
                        out_hbm.at[cid, pl.ds(r0, ROWS_PER_TILE)])

    return _sc_scatter


_TC_R = 1000


def _tc_body(x_ref, p0_ref, p1_ref, wt_ref, b_ref, o_ref):
    s = p0_ref[0, :, :D_IN] + p1_ref[0, :, :D_IN]
    deg = p0_ref[0, :, D_IN:D_IN + 1] + p1_ref[0, :, D_IN:D_IN + 1]
    agg = s / jnp.maximum(deg, 1.0)
    out = jnp.dot(x_ref[...], wt_ref[:D_IN, :],
                  preferred_element_type=jnp.float32)
    out += jnp.dot(agg, wt_ref[D_IN:, :], preferred_element_type=jnp.float32)
    o_ref[...] = out + b_ref[...]


def kernel(x, edge_index, W, b):
    ei = edge_index.astype(jnp.int32)
    pad = E_TOTAL_PAD - N_EDGES
    row = jnp.concatenate(
        [ei[0], jnp.full((pad,), DUMMY_ROW, jnp.int32)]).reshape(
            NW, NB, EDGE_B)
    col = jnp.concatenate(
        [ei[1], jnp.zeros((pad,), jnp.int32)]).reshape(NW, NB, EDGE_B)
    ones_pad = jnp.concatenate(
        [jnp.ones((N_NODES, 1), jnp.float32),
         jnp.zeros((N_NODES, DP - D_IN - 1), jnp.float32)], axis=1)
    xa = jnp.concatenate([x.astype(jnp.float32), ones_pad], axis=1)
    zeros = jnp.zeros((ROWS_PER_TILE, DP), jnp.float32)

    partials = _build_sc_scatter()(xa, col, row, zeros)

    wt = W.T.astype(jnp.float32)
    b2 = b.reshape(1, D_OUT).astype(jnp.float32)
    grid = (N_NODES // _TC_R,)
    return pl.pallas_call(
        _tc_body,
        grid=grid,
        in_specs=[
            pl.BlockSpec((_TC_R, D_IN), lambda i: (i, 0)),
            pl.BlockSpec((1, _TC_R, DP), lambda i: (0, i, 0)),
            pl.BlockSpec((1, _TC_R, DP), lambda i: (1, i, 0)),
            pl.BlockSpec((2 * D_IN, D_OUT), lambda i: (0, 0)),
            pl.BlockSpec((1, D_OUT), lambda i: (0, 0)),
        ],
        out_specs=pl.BlockSpec((_TC_R, D_OUT), lambda i: (i, 0)),
        out_shape=jax.ShapeDtypeStruct((N_NODES, D_OUT), jnp.float32),
    )(x.astype(jnp.float32), partials, partials, wt, b2)
